# Initial kernel scaffold; baseline (speedup 1.0000x reference)
import jax
import jax.numpy as jnp
from jax.experimental import pallas as pl


def kernel(query, key, value, edges, batch, w_q, w_k, w_v, b_q, b_k, b_v, w_out, b_out):
    raise NotImplementedError("write your pallas kernel here")


# speedup vs baseline: 22.6238x; 22.6238x over previous
import functools
import math

import numpy as np
import jax
import jax.numpy as jnp
from jax import lax
from jax.experimental import pallas as pl
from jax.experimental.pallas import tpu as pltpu
from jax.experimental.pallas import tpu_sc as plsc

NUM_HEADS = 8
HEAD_DIM = 16
EMBED = NUM_HEADS * HEAD_DIM
NC = 2
NS = 16
LANES = 16
ACC_W = EMBED + LANES

_HP = jax.lax.Precision.HIGHEST

_S_np = np.zeros((NUM_HEADS, EMBED), np.float32)
for _h in range(NUM_HEADS):
    _S_np[_h, _h * HEAD_DIM:(_h + 1) * HEAD_DIM] = 1.0


def _proj_body(xq, xk, xv, wq, wk, wv, bq, bk, bv, qo, ko, vo):
    qo[...] = (jnp.dot(xq[...], wq[...], precision=_HP) + bq[...]) * (
        1.0 / math.sqrt(HEAD_DIM))
    ko[...] = jnp.dot(xk[...], wk[...], precision=_HP) + bk[...]
    vo[...] = jnp.dot(xv[...], wv[...], precision=_HP) + bv[...]


def _project(query, key, value, wqT, wkT, wvT, bq, bk, bv):
    n, d = query.shape
    bn = 2000
    row = pl.BlockSpec((bn, d), lambda i: (i, 0))
    full = pl.BlockSpec((d, d), lambda i: (0, 0))
    bias = pl.BlockSpec((1, d), lambda i: (0, 0))
    return pl.pallas_call(
        _proj_body,
        grid=(n // bn,),
        in_specs=[row, row, row, full, full, full, bias, bias, bias],
        out_specs=[row, row, row],
        out_shape=[jax.ShapeDtypeStruct((n, d), jnp.float32)] * 3,
    )(query, key, value, wqT, wkT, wvT, bq, bk, bv)


def _out_body(num, den8, S, wT, b, o):
    acc = num[0] + num[1]
    den = jnp.dot(den8[0] + den8[1], S[...], precision=_HP)
    o[...] = jnp.dot(acc / (den + 1e-8), wT[...], precision=_HP) + b[...]


def _out_proj(nd_num, nd_den8, S, w_outT, b_out):
    n = nd_num.shape[1]
    bn = 2000
    return pl.pallas_call(
        _out_body,
        grid=(n // bn,),
        in_specs=[
            pl.BlockSpec((NC, bn, EMBED), lambda i: (0, i, 0)),
            pl.BlockSpec((NC, bn, NUM_HEADS), lambda i: (0, i, 0)),
            pl.BlockSpec((NUM_HEADS, EMBED), lambda i: (0, 0)),
            pl.BlockSpec((EMBED, EMBED), lambda i: (0, 0)),
            pl.BlockSpec((1, EMBED), lambda i: (0, 0)),
        ],
        out_specs=pl.BlockSpec((bn, EMBED), lambda i: (i, 0)),
        out_shape=jax.ShapeDtypeStruct((n, EMBED), jnp.float32),
    )(nd_num, nd_den8, S, w_outT, b_out)


def _sc_edge_attention(q, k, v, e0, e1):
    n = q.shape[0]
    m = e0.shape[0]
    W = NC * NS
    EC = 32
    total_chunks = m // EC
    base_chunks = total_chunks // W
    extra_chunks = total_chunks - base_chunks * W
    n_pad = ((n + 2047) // 2048) * 2048
    rows_per_tile = n_pad // NS
    den_rows = (n_pad + 7) // 8
    acc_rows = n_pad + den_rows
    zac = acc_rows // NS
    mesh = plsc.VectorSubcoreMesh(core_axis_name="c", subcore_axis_name="s")

    @functools.partial(
        pl.kernel,
        mesh=mesh,
        out_type=[
            jax.ShapeDtypeStruct((NC, NS, rows_per_tile, EMBED), jnp.float32),
            jax.ShapeDtypeStruct((NC, NS, den_rows // NS, EMBED), jnp.float32),
        ],
        scratch_types=[
            pltpu.VMEM((EC,), jnp.int32),
            pltpu.VMEM((EC,), jnp.int32),
            pltpu.VMEM((EC,), jnp.int32),
            pltpu.VMEM((EC, EMBED), jnp.float32),
            pltpu.VMEM((EC, EMBED), jnp.float32),
            pltpu.VMEM((EC, EMBED), jnp.float32),
            pltpu.VMEM((EC, EMBED), jnp.float32),
            pltpu.VMEM((EC, EMBED), jnp.float32),
            pltpu.VMEM_SHARED((acc_rows, EMBED), jnp.float32),
            pltpu.SemaphoreType.DMA,
            pltpu.SemaphoreType.DMA,
            pltpu.SemaphoreType.DMA,
        ],
    )
    def sc_kern(q_hbm, k_hbm, v_hbm, e0_hbm, e1_hbm, num_hbm, den_hbm,
                e0_c, e1_c, didx_v, q_rows, k_rows, v_rows, msg_v,
                den_msg, acc_sh, sem_q, sem_k, sem_v):
        cid = lax.axis_index("c")
        sid = lax.axis_index("s")
        iota = lax.iota(jnp.int32, LANES)
        zvec = jnp.zeros((LANES,), jnp.float32)

        def zrow(ref):
            def body(r, carry):
                for cblk in range(ref.shape[1] // LANES):
                    ref[r, pl.ds(cblk * LANES, LANES)] = zvec
                return carry
            lax.fori_loop(0, ref.shape[0], body, 0)

        zrow(msg_v)
        zrow(den_msg)
        z0 = sid * zac
        for j in range(zac // EC):
            pltpu.sync_copy(msg_v, acc_sh.at[pl.ds(z0 + j * EC, EC)])
        for j in range(zac // EC * EC, zac, 8):
            pltpu.sync_copy(msg_v.at[pl.ds(0, 8)],
                            acc_sh.at[pl.ds(z0 + j, 8)])
        plsc.subcore_barrier()

        _gdn = lax.GatherDimensionNumbers(
            offset_dims=(), collapsed_slice_dims=(0,), start_index_map=(0,))

        def _lane_perm(x, prm):
            return lax.gather(
                x, prm[:, None], dimension_numbers=_gdn, slice_sizes=(1,),
                mode=lax.GatherScatterMode.PROMISE_IN_BOUNDS)

        perms = [iota ^ jnp.int32(stp) for stp in (8, 4, 2, 1)]
        wid = cid * NS + sid
        my_chunks = jnp.where(wid < extra_chunks,
                              base_chunks + 1, base_chunks)
        chunk0 = base_chunks * wid + jnp.minimum(wid, extra_chunks)

        def chunk_body(ci, carry):
            base = (chunk0 + ci) * EC
            pltpu.sync_copy(e0_hbm.at[pl.ds(base, EC)], e0_c)
            pltpu.sync_copy(e1_hbm.at[pl.ds(base, EC)], e1_c)
            cq = pltpu.async_copy(q_hbm.at[e0_c], q_rows, sem_q)
            ck = pltpu.async_copy(k_hbm.at[e1_c], k_rows, sem_k)
            cv = pltpu.async_copy(v_hbm.at[e1_c], v_rows, sem_v)
            cq.wait()
            ck.wait()
            cv.wait()

            for g in range(EC // LANES):
                g16 = g * LANES
                e0g = e0_c[pl.ds(g16, LANES)]
                didx_v[pl.ds(g16, LANES)] = (
                    jnp.int32(n_pad) + lax.shift_right_logical(e0g, 3))
                for el in range(LANES):
                    e = g16 + el
                    attn_vec = jnp.zeros((LANES,), jnp.float32)
                    for h in range(NUM_HEADS):
                        sl = pl.ds(h * HEAD_DIM, HEAD_DIM)
                        p = q_rows[e, sl] * k_rows[e, sl]
                        for prm in perms:
                            p = p + _lane_perm(p, prm)
                        av = jnp.exp(p)
                        msg_v[e, sl] = av * v_rows[e, sl]
                        attn_vec = jnp.where(iota == h, av, attn_vec)
                    dcol = (e0g[el] & 7) * LANES
                    den_msg[e, pl.ds(dcol, LANES)] = attn_vec

            pltpu.sync_copy(msg_v, acc_sh.at[e0_c], add=True)
            pltpu.sync_copy(den_msg, acc_sh.at[didx_v], add=True)

            for g in range(EC // LANES):
                g16 = g * LANES
                e0g = e0_c[pl.ds(g16, LANES)]
                for el in range(LANES):
                    dcol = (e0g[el] & 7) * LANES
                    den_msg[g16 + el, pl.ds(dcol, LANES)] = zvec
            return carry

        lax.fori_loop(0, my_chunks, chunk_body, 0)
        plsc.subcore_barrier()

        r_base = sid * rows_per_tile
        for j in range(rows_per_tile // EC):
            r0 = r_base + j * EC
            pltpu.sync_copy(acc_sh.at[pl.ds(r0, EC)], q_rows)
            pltpu.sync_copy(q_rows, num_hbm.at[cid, sid, pl.ds(j * EC, EC)])
        den_rpt = den_rows // NS
        d_base = n_pad + sid * den_rpt
        for j in range(den_rpt // EC):
            d0 = d_base + j * EC
            pltpu.sync_copy(acc_sh.at[pl.ds(d0, EC)], q_rows)
            pltpu.sync_copy(q_rows, den_hbm.at[cid, sid, pl.ds(j * EC, EC)])
        for j in range(den_rpt // EC * EC, den_rpt, 8):
            pltpu.sync_copy(acc_sh.at[pl.ds(d_base + j, 8)],
                            q_rows.at[pl.ds(0, 8)])
            pltpu.sync_copy(q_rows.at[pl.ds(0, 8)],
                            den_hbm.at[cid, sid, pl.ds(j, 8)])

    num, den = sc_kern(q, k, v, e0, e1)
    return (num.reshape(NC, n_pad, EMBED),
            den.reshape(NC, den_rows, EMBED))


def kernel(query, key, value, edges, batch, w_q, w_k, w_v, b_q, b_k, b_v,
           w_out, b_out):
    del batch
    f32 = jnp.float32
    e0 = edges[0].astype(jnp.int32)
    e1 = edges[1].astype(jnp.int32)
    q, k, v = _project(
        query.astype(f32), key.astype(f32), value.astype(f32),
        w_q.T.astype(f32), w_k.T.astype(f32), w_v.T.astype(f32),
        b_q.reshape(1, -1).astype(f32), b_k.reshape(1, -1).astype(f32),
        b_v.reshape(1, -1).astype(f32))
    n = query.shape[0]
    nd_num, nd_den = _sc_edge_attention(q, k, v, e0, e1)
    nd_num = nd_num[:, :n, :]
    nd_den8 = nd_den.reshape(NC, -1, LANES)[:, :n, :NUM_HEADS]
    S = jnp.asarray(_S_np)
    return _out_proj(nd_num, nd_den8, S, w_out.T.astype(f32),
                     b_out.reshape(1, -1).astype(f32))


# speedup vs baseline: 27.8316x; 1.2302x over previous
import functools
import math

import numpy as np
import jax
import jax.numpy as jnp
from jax import lax
from jax.experimental import pallas as pl
from jax.experimental.pallas import tpu as pltpu
from jax.experimental.pallas import tpu_sc as plsc

NUM_HEADS = 8
HEAD_DIM = 16
EMBED = NUM_HEADS * HEAD_DIM
NC = 2
NS = 16
LANES = 16
ACC_W = EMBED + LANES

_HP = jax.lax.Precision.HIGHEST

_S_np = np.zeros((NUM_HEADS, EMBED), np.float32)
for _h in range(NUM_HEADS):
    _S_np[_h, _h * HEAD_DIM:(_h + 1) * HEAD_DIM] = 1.0


def _proj_body(xq, xk, xv, wq, wk, wv, bq, bk, bv, qo, ko, vo):
    qo[...] = (jnp.dot(xq[...], wq[...], precision=_HP) + bq[...]) * (
        1.0 / math.sqrt(HEAD_DIM))
    ko[...] = jnp.dot(xk[...], wk[...], precision=_HP) + bk[...]
    vo[...] = jnp.dot(xv[...], wv[...], precision=_HP) + bv[...]


def _project(query, key, value, wqT, wkT, wvT, bq, bk, bv):
    n, d = query.shape
    bn = 2000
    row = pl.BlockSpec((bn, d), lambda i: (i, 0))
    full = pl.BlockSpec((d, d), lambda i: (0, 0))
    bias = pl.BlockSpec((1, d), lambda i: (0, 0))
    return pl.pallas_call(
        _proj_body,
        grid=(n // bn,),
        in_specs=[row, row, row, full, full, full, bias, bias, bias],
        out_specs=[row, row, row],
        out_shape=[jax.ShapeDtypeStruct((n, d), jnp.float32)] * 3,
    )(query, key, value, wqT, wkT, wvT, bq, bk, bv)


def _out_body(num, den8, S, wT, b, o):
    acc = num[0] + num[1]
    den = jnp.dot(den8[0] + den8[1], S[...], precision=_HP)
    o[...] = jnp.dot(acc / (den + 1e-8), wT[...], precision=_HP) + b[...]


def _out_proj(nd_num, nd_den8, S, w_outT, b_out):
    n = nd_num.shape[1]
    bn = 2000
    return pl.pallas_call(
        _out_body,
        grid=(n // bn,),
        in_specs=[
            pl.BlockSpec((NC, bn, EMBED), lambda i: (0, i, 0)),
            pl.BlockSpec((NC, bn, NUM_HEADS), lambda i: (0, i, 0)),
            pl.BlockSpec((NUM_HEADS, EMBED), lambda i: (0, 0)),
            pl.BlockSpec((EMBED, EMBED), lambda i: (0, 0)),
            pl.BlockSpec((1, EMBED), lambda i: (0, 0)),
        ],
        out_specs=pl.BlockSpec((bn, EMBED), lambda i: (i, 0)),
        out_shape=jax.ShapeDtypeStruct((n, EMBED), jnp.float32),
    )(nd_num, nd_den8, S, w_outT, b_out)


def _sc_edge_attention(q, k, v, e0, e1):
    n = q.shape[0]
    m = e0.shape[0]
    W = NC * NS
    EC = 32
    total_chunks = m // EC
    base_chunks = total_chunks // W
    extra_chunks = total_chunks - base_chunks * W
    n_pad = ((n + 2047) // 2048) * 2048
    rows_per_tile = n_pad // NS
    den_rows = (n_pad + 7) // 8
    acc_rows = n_pad + den_rows
    zac = acc_rows // NS
    mesh = plsc.VectorSubcoreMesh(core_axis_name="c", subcore_axis_name="s")

    @functools.partial(
        pl.kernel,
        mesh=mesh,
        out_type=[
            jax.ShapeDtypeStruct((NC, NS, rows_per_tile, EMBED), jnp.float32),
            jax.ShapeDtypeStruct((NC, NS, den_rows // NS, EMBED), jnp.float32),
        ],
        scratch_types=[
            pltpu.VMEM((2, EC), jnp.int32),
            pltpu.VMEM((2, EC), jnp.int32),
            pltpu.VMEM((2, EC), jnp.int32),
            pltpu.VMEM((EC, EMBED), jnp.float32),
            pltpu.VMEM((EC, EMBED), jnp.float32),
            pltpu.VMEM((EC, EMBED), jnp.float32),
            pltpu.VMEM((EC, EMBED), jnp.float32),
            pltpu.VMEM((EC, EMBED), jnp.float32),
            pltpu.VMEM_SHARED((acc_rows, EMBED), jnp.float32),
            pltpu.SemaphoreType.DMA,
            pltpu.SemaphoreType.DMA,
            pltpu.SemaphoreType.DMA,
            pltpu.SemaphoreType.DMA,
            pltpu.SemaphoreType.DMA,
        ],
    )
    def sc_kern(q_hbm, k_hbm, v_hbm, e0_hbm, e1_hbm, num_hbm, den_hbm,
                e0_c2, e1_c2, didx2, q_rows, k_rows, v_rows, msg_v,
                den_msg, acc_sh, sem_q, sem_k, sem_v, sem_s1, sem_s2):
        cid = lax.axis_index("c")
        sid = lax.axis_index("s")
        iota = lax.iota(jnp.int32, LANES)
        zvec = jnp.zeros((LANES,), jnp.float32)

        def zrow(ref):
            def body(r, carry):
                for cblk in range(ref.shape[1] // LANES):
                    ref[r, pl.ds(cblk * LANES, LANES)] = zvec
                return carry
            lax.fori_loop(0, ref.shape[0], body, 0)

        zrow(msg_v)
        zrow(den_msg)
        z0 = sid * zac
        for j in range(zac // EC):
            pltpu.sync_copy(msg_v, acc_sh.at[pl.ds(z0 + j * EC, EC)])
        for j in range(zac // EC * EC, zac, 8):
            pltpu.sync_copy(msg_v.at[pl.ds(0, 8)],
                            acc_sh.at[pl.ds(z0 + j, 8)])
        plsc.subcore_barrier()

        _gdn = lax.GatherDimensionNumbers(
            offset_dims=(), collapsed_slice_dims=(0,), start_index_map=(0,))

        def _lane_perm(x, prm):
            return lax.gather(
                x, prm[:, None], dimension_numbers=_gdn, slice_sizes=(1,),
                mode=lax.GatherScatterMode.PROMISE_IN_BOUNDS)

        perms = [iota ^ jnp.int32(stp) for stp in (8, 4, 2, 1)]
        wid = cid * NS + sid
        my_chunks = jnp.where(wid < extra_chunks,
                              base_chunks + 1, base_chunks)
        chunk0 = base_chunks * wid + jnp.minimum(wid, extra_chunks)

        def load_idx(ci, slot):
            base = (chunk0 + ci) * EC
            pltpu.sync_copy(e0_hbm.at[pl.ds(base, EC)], e0_c2.at[slot])
            pltpu.sync_copy(e1_hbm.at[pl.ds(base, EC)], e1_c2.at[slot])

        load_idx(jnp.int32(0), jnp.int32(0))

        def chunk_body(ci, carry):
            p = ci & 1
            pp = 1 - p
            cq = pltpu.async_copy(q_hbm.at[e0_c2.at[p]], q_rows, sem_q)
            ck = pltpu.async_copy(k_hbm.at[e1_c2.at[p]], k_rows, sem_k)
            cv = pltpu.async_copy(v_hbm.at[e1_c2.at[p]], v_rows, sem_v)

            @pl.when(ci > 0)
            def _wait_prev():
                pltpu.make_async_copy(
                    msg_v, acc_sh.at[e0_c2.at[pp]], sem_s1).wait()
                pltpu.make_async_copy(
                    den_msg, acc_sh.at[didx2.at[pp]], sem_s2).wait()
                for g in range(EC // LANES):
                    g16 = g * LANES
                    e0g = e0_c2[pp, pl.ds(g16, LANES)]
                    for el in range(LANES):
                        dcol = (e0g[el] & 7) * LANES
                        den_msg[g16 + el, pl.ds(dcol, LANES)] = zvec

            load_idx(jnp.minimum(ci + 1, my_chunks - 1), pp)
            cq.wait()
            ck.wait()
            cv.wait()

            for g in range(EC // LANES):
                g16 = g * LANES
                e0g = e0_c2[p, pl.ds(g16, LANES)]
                didx2[p, pl.ds(g16, LANES)] = (
                    jnp.int32(n_pad) + lax.shift_right_logical(e0g, 3))
                for el in range(LANES):
                    e = g16 + el
                    attn_vec = jnp.zeros((LANES,), jnp.float32)
                    for h in range(NUM_HEADS):
                        sl = pl.ds(h * HEAD_DIM, HEAD_DIM)
                        p_qk = q_rows[e, sl] * k_rows[e, sl]
                        for prm in perms:
                            p_qk = p_qk + _lane_perm(p_qk, prm)
                        av = jnp.exp(p_qk)
                        msg_v[e, sl] = av * v_rows[e, sl]
                        attn_vec = jnp.where(iota == h, av, attn_vec)
                    dcol = (e0g[el] & 7) * LANES
                    den_msg[e, pl.ds(dcol, LANES)] = attn_vec

            pltpu.async_copy(msg_v, acc_sh.at[e0_c2.at[p]], sem_s1, add=True)
            pltpu.async_copy(den_msg, acc_sh.at[didx2.at[p]], sem_s2,
                             add=True)
            return carry

        lax.fori_loop(0, my_chunks, chunk_body, 0)
        p_last = (my_chunks - 1) & 1
        pltpu.make_async_copy(
            msg_v, acc_sh.at[e0_c2.at[p_last]], sem_s1).wait()
        pltpu.make_async_copy(
            den_msg, acc_sh.at[didx2.at[p_last]], sem_s2).wait()
        plsc.subcore_barrier()

        r_base = sid * rows_per_tile
        for j in range(rows_per_tile // EC):
            r0 = r_base + j * EC
            pltpu.sync_copy(acc_sh.at[pl.ds(r0, EC)], q_rows)
            pltpu.sync_copy(q_rows, num_hbm.at[cid, sid, pl.ds(j * EC, EC)])
        den_rpt = den_rows // NS
        d_base = n_pad + sid * den_rpt
        for j in range(den_rpt // EC):
            d0 = d_base + j * EC
            pltpu.sync_copy(acc_sh.at[pl.ds(d0, EC)], q_rows)
            pltpu.sync_copy(q_rows, den_hbm.at[cid, sid, pl.ds(j * EC, EC)])
        for j in range(den_rpt // EC * EC, den_rpt, 8):
            pltpu.sync_copy(acc_sh.at[pl.ds(d_base + j, 8)],
                            q_rows.at[pl.ds(0, 8)])
            pltpu.sync_copy(q_rows.at[pl.ds(0, 8)],
                            den_hbm.at[cid, sid, pl.ds(j, 8)])

    num, den = sc_kern(q, k, v, e0, e1)
    return (num.reshape(NC, n_pad, EMBED),
            den.reshape(NC, den_rows, EMBED))


def kernel(query, key, value, edges, batch, w_q, w_k, w_v, b_q, b_k, b_v,
           w_out, b_out):
    del batch
    f32 = jnp.float32
    e0 = edges[0].astype(jnp.int32)
    e1 = edges[1].astype(jnp.int32)
    q, k, v = _project(
        query.astype(f32), key.astype(f32), value.astype(f32),
        w_q.T.astype(f32), w_k.T.astype(f32), w_v.T.astype(f32),
        b_q.reshape(1, -1).astype(f32), b_k.reshape(1, -1).astype(f32),
        b_v.reshape(1, -1).astype(f32))
    n = query.shape[0]
    nd_num, nd_den = _sc_edge_attention(q, k, v, e0, e1)
    nd_num = nd_num[:, :n, :]
    nd_den8 = nd_den.reshape(NC, -1, LANES)[:, :n, :NUM_HEADS]
    S = jnp.asarray(_S_np)
    return _out_proj(nd_num, nd_den8, S, w_out.T.astype(f32),
                     b_out.reshape(1, -1).astype(f32))
